# baseline (speedup 1.0000x reference)
import functools

import jax
import jax.numpy as jnp
import numpy as np
from jax import lax
from jax.experimental import pallas as pl
from jax.experimental.pallas import tpu as pltpu
from jax.experimental.pallas import tpu_sc as plsc

N = 100000
E = 1600000
HID = 64

NC = 2
NS = 16
NW = NC * NS
L = 16

_CP = pltpu.CompilerParams(use_tc_tiling_on_sc=False)
_CP_NL = pltpu.CompilerParams(use_tc_tiling_on_sc=False,
                              needs_layout_passes=False)

BK = 1 << 14
NB = (N + BK - 1) // BK
NROWS = NB * BK
DUMP = BK

E_PER_W = E // NW
SUB = 10000
NSUB = E_PER_W // SUB
PBUF = 50688
CAPW = 50688
CH = 128
GZ = 3
SEG_ALIGN = GZ * CH

PADQ = np.int32(np.uint32(N | (np.uint32(DUMP) << 17)))

NPAD = 100352
RB = 2048
GRID = NPAD // RB

_SC_COST = pl.CostEstimate(flops=0, transcendentals=0,
                           bytes_accessed=E * 4)


def _bin_body(ei_hbm, slab_hbm, counts_hbm, src_v, dst_v, pbuf, cnt_v, sem):
    c = lax.axis_index("c")
    s = lax.axis_index("s")
    wid = s * NC + c
    base = wid * E_PER_W
    counts_vec = jnp.zeros((L,), jnp.int32)
    padv = jnp.full((L,), PADQ, jnp.int32)
    for b in range(NB):
        def subchunk(k, cnt, b=b):
            off = base + k * SUB
            pltpu.sync_copy(ei_hbm.at[0, pl.ds(off, SUB)], src_v)
            pltpu.sync_copy(ei_hbm.at[1, pl.ds(off, SUB)], dst_v)

            def vec(i, cnt):
                sv = src_v[pl.ds(i * L, L)]
                dv = dst_v[pl.ds(i * L, L)]
                m = (dv >> 14) == b
                q = sv | ((dv & (BK - 1)) << 17)
                plsc.store_compressed(pbuf.at[pl.ds(cnt, L)], q, mask=m)
                return cnt + jnp.sum(m.astype(jnp.int32))

            return lax.fori_loop(0, SUB // L, vec, cnt)

        cnt = lax.fori_loop(0, NSUB, subchunk, jnp.int32(0))
        pbuf[pl.ds(cnt, L)] = padv
        cnt = (cnt + 15) & jnp.int32(-16)

        def wbody(cc):
            pbuf[pl.ds(cc, L)] = padv
            return cc + L

        cnt = lax.while_loop(lambda cc: (cc % SEG_ALIGN) != 0, wbody, cnt)
        counts_vec = counts_vec + cnt * (lax.iota(jnp.int32, L) == b)

        nbig = cnt >> 11

        def fbig(j, _, b=b):
            pltpu.sync_copy(pbuf.at[pl.ds(j * 2048, 2048)],
                            slab_hbm.at[b, wid, pl.ds(j * 2048, 2048)])
            return 0

        lax.fori_loop(0, nbig, fbig, 0)

        def fsm(j, _, b=b):
            pltpu.sync_copy(pbuf.at[pl.ds(j * 128, 128)],
                            slab_hbm.at[b, wid, pl.ds(j * 128, 128)])
            return 0

        lax.fori_loop(nbig * L, cnt >> 7, fsm, 0)
    cnt_v[...] = counts_vec
    pltpu.sync_copy(cnt_v, counts_hbm.at[wid])


def _bin_kernel(mesh):
  return functools.partial(
    pl.kernel, _bin_body, mesh=mesh, compiler_params=_CP_NL,
    out_type=[jax.ShapeDtypeStruct((NB, NW, CAPW), jnp.int32),
              jax.ShapeDtypeStruct((NW, L), jnp.int32)],
    scratch_types=[pltpu.VMEM((SUB,), jnp.int32),
                   pltpu.VMEM((SUB,), jnp.int32),
                   pltpu.VMEM((PBUF,), jnp.int32),
                   pltpu.VMEM((L,), jnp.int32),
                   pltpu.SemaphoreType.DMA],
    cost_estimate=_SC_COST, name="gcn_bin")()


DGB = 2048


def _deg_body(slab_hbm, counts_hbm, deg_hbm, qbuf, dstl2, dtail_v, ones_v,
              zero_v, cvec_v, dacc, sem):
    c = lax.axis_index("c")
    s = lax.axis_index("s")

    @pl.loop(0, CH)
    def _(r):
        ones_v[r, pl.ds(0, L)] = jnp.full((L,), 1.0, jnp.float32)
        zero_v[r, pl.ds(0, L)] = jnp.zeros((L,), jnp.float32)

    for b in range(NB):
        @pl.when((b % NC) == c)
        def _(b=b):
            @pl.loop(0, BK // NS // 128)
            def _(j):
                pltpu.sync_copy(
                    zero_v, dacc.at[pl.ds(s * (BK // NS) + j * 128, 128)])
            plsc.subcore_barrier()
            for k in range(2):
                wseg = s * 2 + k
                pltpu.sync_copy(counts_hbm.at[wseg], cvec_v)
                n = cvec_v[...][b]

                def bulk(j, _, b=b, wseg=wseg):
                    pltpu.sync_copy(slab_hbm.at[b, wseg, pl.ds(j * DGB, DGB)],
                                    qbuf)

                    @pl.loop(0, DGB // CH)
                    def _(jj):
                        @pl.loop(0, CH // L)
                        def _(i):
                            dstl2[jj, pl.ds(i * L, L)] = (
                                lax.shift_right_logical(
                                    qbuf[pl.ds(jj * CH + i * L, L)], 17))

                    @pl.loop(0, DGB // CH)
                    def _(jj):
                        pltpu.sync_copy(ones_v, dacc.at[dstl2.at[jj]],
                                        add=True)
                    return 0

                nbig = n // DGB
                lax.fori_loop(0, nbig, bulk, 0)

                def tail(j, _, b=b, wseg=wseg):
                    pltpu.sync_copy(
                        slab_hbm.at[b, wseg, pl.ds(j * CH, CH)],
                        qbuf.at[pl.ds(0, CH)])

                    @pl.loop(0, CH // L)
                    def _(i):
                        dtail_v[pl.ds(i * L, L)] = lax.shift_right_logical(
                            qbuf[pl.ds(i * L, L)], 17)

                    pltpu.sync_copy(ones_v, dacc.at[dtail_v], add=True)
                    return 0

                lax.fori_loop(nbig * (DGB // CH), n // CH, tail, 0)
            plsc.subcore_barrier()
            pltpu.sync_copy(
                dacc.at[pl.ds(s * (BK // NS), BK // NS)],
                deg_hbm.at[pl.ds(b * BK + s * (BK // NS), BK // NS)])


def _deg_kernel(mesh):
  return functools.partial(
    pl.kernel, _deg_body, mesh=mesh, compiler_params=_CP,
    out_type=jax.ShapeDtypeStruct((NROWS, L), jnp.float32),
    scratch_types=[pltpu.VMEM((DGB,), jnp.int32),
                   pltpu.VMEM((DGB // CH, CH), jnp.int32),
                   pltpu.VMEM((CH,), jnp.int32),
                   pltpu.VMEM((CH, L), jnp.float32),
                   pltpu.VMEM((128, L), jnp.float32),
                   pltpu.VMEM((L,), jnp.int32),
                   pltpu.VMEM_SHARED((BK + L, L), jnp.float32),
                   pltpu.SemaphoreType.DMA],
    cost_estimate=_SC_COST, name="gcn_deg")()


def _acc_body(slab_hbm, counts_hbm, g_hbm, s_hbm, qb2, s2, d2, rall2,
              zero_v, cvec_v, acc, semg, sems):
    c = lax.axis_index("c")
    s = lax.axis_index("s")

    @pl.loop(0, 128)
    def _(r):
        for j in range(HID // L):
            zero_v[r, pl.ds(j * L, L)] = jnp.zeros((L,), jnp.float32)

    def load_unpack(t, p, b, wseg):
        pltpu.sync_copy(
            slab_hbm.at[b, wseg, pl.ds(t * SEG_ALIGN, SEG_ALIGN)],
            qb2.at[p])
        for jj in range(GZ):
            @pl.loop(0, CH // L)
            def _(i):
                q = qb2[p, pl.ds(jj * CH + i * L, L)]
                s2[p * GZ + jj, pl.ds(i * L, L)] = q & 0x1FFFF
                d2[p * GZ + jj, pl.ds(i * L, L)] = lax.shift_right_logical(
                    q, 17)

    def gather_descs(p):
        return [pltpu.make_async_copy(
                    g_hbm.at[s2.at[p * GZ + jj]],
                    rall2.at[pl.ds((p * GZ + jj) * CH, CH)],
                    semg.at[p * GZ + jj])
                for jj in range(GZ)]

    def scatter_descs(p):
        return [(rall2.at[pl.ds((p * GZ + jj) * CH, CH)],
                 acc.at[d2.at[p * GZ + jj]], sems.at[p * GZ + jj])
                for jj in range(GZ)]

    for b in range(NB):
        @pl.when((b % NC) == c)
        def _(b=b):
            @pl.loop(0, BK // NS // 128)
            def _(j):
                pltpu.sync_copy(
                    zero_v, acc.at[pl.ds(s * (BK // NS) + j * 128, 128)])
            plsc.subcore_barrier()
            for k in range(2):
                wseg = s * 2 + k
                pltpu.sync_copy(counts_hbm.at[wseg], cvec_v)
                n = cvec_v[...][b]
                ngr = n // SEG_ALIGN

                def body(t, _, b=b, wseg=wseg, ngr=ngr):
                    for p in range(2):
                        @pl.when((t & 1) == p)
                        def _(p=p):
                            @pl.when(t >= 2)
                            def _():
                                for sr, dst, sm in scatter_descs(p):
                                    pltpu.make_async_copy(sr, dst, sm).wait()

                            @pl.when(t < ngr)
                            def _():
                                load_unpack(t, p, b, wseg)
                                for de in gather_descs(p):
                                    de.start()

                            @pl.when((t >= 1) & ((t - 1) < ngr))
                            def _():
                                for de in gather_descs(1 - p):
                                    de.wait()
                                for sr, dst, sm in scatter_descs(1 - p):
                                    pltpu.async_copy(sr, dst, sm, add=True)
                    return 0

                lax.fori_loop(0, ngr + 2, body, 0)
            plsc.subcore_barrier()
            pltpu.sync_copy(
                acc.at[pl.ds(s * (BK // NS), BK // NS)],
                s_hbm.at[pl.ds(b * BK + s * (BK // NS), BK // NS)])


def _acc_kernel(mesh):
  return functools.partial(
    pl.kernel, _acc_body, mesh=mesh, compiler_params=_CP,
    out_type=jax.ShapeDtypeStruct((NROWS, HID), jnp.float32),
    scratch_types=[pltpu.VMEM((2, SEG_ALIGN), jnp.int32),
                   pltpu.VMEM((2 * GZ, CH), jnp.int32),
                   pltpu.VMEM((2 * GZ, CH), jnp.int32),
                   pltpu.VMEM((2 * SEG_ALIGN, HID), jnp.float32),
                   pltpu.VMEM((128, HID), jnp.float32),
                   pltpu.VMEM((L,), jnp.int32),
                   pltpu.VMEM_SHARED((BK + L, HID), jnp.float32),
                   pltpu.SemaphoreType.DMA((2 * GZ,)),
                   pltpu.SemaphoreType.DMA((2 * GZ,))],
    cost_estimate=_SC_COST, name="gcn_acc")()


def _t1_body(x_ref, deg_ref, w_ref, out_ref):
    dinv = lax.rsqrt(deg_ref[:, 0:1] + 1.0)
    xb = x_ref[...]
    w = w_ref[...]
    h = (xb[:, 0:1] * w[0:1, :] + xb[:, 1:2] * w[1:2, :]
         + xb[:, 2:3] * w[2:3, :])
    out_ref[...] = dinv * h


def _tmid_body(s_ref, g_ref, deg_ref, w_ref, b_ref, out_ref):
    dinv = lax.rsqrt(deg_ref[:, 0:1] + 1.0)
    t = jnp.maximum(dinv * (s_ref[...] + g_ref[...]) + b_ref[0:1, :], 0.0)
    out_ref[...] = dinv * jnp.dot(t, w_ref[...],
                                  preferred_element_type=jnp.float32)


def _t4_body(s_ref, g_ref, deg_ref, b3_ref, m1_ref, mb1_ref, m2_ref, mb2_ref,
             out_ref):
    dinv = lax.rsqrt(deg_ref[:, 0:1] + 1.0)
    h3 = jnp.maximum(dinv * (s_ref[...] + g_ref[...]) + b3_ref[0:1, :], 0.0)
    h4 = jnp.maximum(jnp.dot(h3, m1_ref[...],
                             preferred_element_type=jnp.float32)
                     + mb1_ref[0:1, :], 0.0)
    z = jnp.sum(h4 * m2_ref[0:1, :], axis=1, keepdims=True) + mb2_ref[0:1, 0:1]
    out_ref[...] = jax.nn.sigmoid(z)


def _row_spec(cols):
    return pl.BlockSpec((RB, cols), lambda i: (i, 0))


def _full_spec(r, cols):
    return pl.BlockSpec((r, cols), lambda i: (0, 0))


_t1_call = pl.pallas_call(
    _t1_body, grid=(GRID,),
    in_specs=[_row_spec(3), _row_spec(L), _full_spec(8, HID)],
    out_specs=_row_spec(HID),
    out_shape=jax.ShapeDtypeStruct((NPAD, HID), jnp.float32))

_tmid_call = pl.pallas_call(
    _tmid_body, grid=(GRID,),
    in_specs=[_row_spec(HID), _row_spec(HID), _row_spec(L),
              _full_spec(HID, HID), _full_spec(8, HID)],
    out_specs=_row_spec(HID),
    out_shape=jax.ShapeDtypeStruct((NPAD, HID), jnp.float32))

_t4_call = pl.pallas_call(
    _t4_body, grid=(GRID,),
    in_specs=[_row_spec(HID), _row_spec(HID), _row_spec(L),
              _full_spec(8, HID), _full_spec(HID, HID), _full_spec(8, HID),
              _full_spec(8, HID), _full_spec(8, HID)],
    out_specs=pl.BlockSpec((RB, 1), lambda i: (i, 0)),
    out_shape=jax.ShapeDtypeStruct((N, 1), jnp.float32))


@functools.lru_cache(maxsize=1)
def _sc_kernels():
    mesh = plsc.VectorSubcoreMesh(core_axis_name="c", subcore_axis_name="s",
                                  num_cores=NC, num_subcores=NS)
    return _bin_kernel(mesh), _deg_kernel(mesh), _acc_kernel(mesh)


def kernel(x, edge_index, W1, b1, W2, b2, W3, b3, M1, mb1, M2, mb2):
    _bin_call, _deg_call, _acc_call = _sc_kernels()
    slab, counts = _bin_call(edge_index)
    deg = _deg_call(slab, counts)

    w1p = jnp.pad(W1, ((0, 5), (0, 0)))
    b1p = jnp.broadcast_to(b1.reshape(1, HID), (8, HID))
    b2p = jnp.broadcast_to(b2.reshape(1, HID), (8, HID))
    b3p = jnp.broadcast_to(b3.reshape(1, HID), (8, HID))
    mb1p = jnp.broadcast_to(mb1.reshape(1, HID), (8, HID))
    m2p = jnp.broadcast_to(M2.reshape(1, HID), (8, HID))
    mb2p = jnp.broadcast_to(mb2.reshape(1, 1), (8, HID))

    g1 = _t1_call(x, deg, w1p)
    s1 = _acc_call(slab, counts, g1)
    g2 = _tmid_call(s1, g1, deg, W2, b1p)
    s2 = _acc_call(slab, counts, g2)
    g3 = _tmid_call(s2, g2, deg, W3, b2p)
    s3 = _acc_call(slab, counts, g3)
    return _t4_call(s3, g3, deg, b3p, M1, mb1p, m2p, mb2p)


# speedup vs baseline: 1.7105x; 1.7105x over previous
import functools

import jax
import jax.numpy as jnp
import numpy as np
from jax import lax
from jax.experimental import pallas as pl
from jax.experimental.pallas import tpu as pltpu
from jax.experimental.pallas import tpu_sc as plsc

N = 100000
E = 1600000
HID = 64

NC = 2
NS = 16
NW = NC * NS
L = 16

_CP = pltpu.CompilerParams(use_tc_tiling_on_sc=False)
_CP_NL = pltpu.CompilerParams(use_tc_tiling_on_sc=False,
                              needs_layout_passes=False)

BK = 1 << 14
NB = (N + BK - 1) // BK
NROWS = NB * BK
DUMP = BK

E_PER_W = E // NW
SUB = 10000
NSUB = E_PER_W // SUB
PBUF = 50688
CAPW = 50688
CH = 128
GACC = 6
SEG_ALIGN = 128

PADQ = np.int32(np.uint32(N | (np.uint32(DUMP) << 17)))

NPAD = 100352
RB = 2048
GRID = NPAD // RB

_SC_COST = pl.CostEstimate(flops=0, transcendentals=0,
                           bytes_accessed=E * 4)


def _bin_body(ei_hbm, slab_hbm, counts_hbm, src_v, dst_v, pbuf, cnt_v, sem):
    c = lax.axis_index("c")
    s = lax.axis_index("s")
    wid = s * NC + c
    base = wid * E_PER_W
    counts_vec = jnp.zeros((L,), jnp.int32)
    padv = jnp.full((L,), PADQ, jnp.int32)
    for b in range(NB):
        def subchunk(k, cnt, b=b):
            off = base + k * SUB
            pltpu.sync_copy(ei_hbm.at[0, pl.ds(off, SUB)], src_v)
            pltpu.sync_copy(ei_hbm.at[1, pl.ds(off, SUB)], dst_v)

            def vec(i, cnt):
                sv = src_v[pl.ds(i * L, L)]
                dv = dst_v[pl.ds(i * L, L)]
                m = (dv >> 14) == b
                q = sv | ((dv & (BK - 1)) << 17)
                plsc.store_compressed(pbuf.at[pl.ds(cnt, L)], q, mask=m)
                return cnt + jnp.sum(m.astype(jnp.int32))

            return lax.fori_loop(0, SUB // L, vec, cnt)

        cnt = lax.fori_loop(0, NSUB, subchunk, jnp.int32(0))
        pbuf[pl.ds(cnt, L)] = padv
        cnt = (cnt + 15) & jnp.int32(-16)

        def wbody(cc):
            pbuf[pl.ds(cc, L)] = padv
            return cc + L

        cnt = lax.while_loop(lambda cc: (cc & 127) != 0, wbody, cnt)
        counts_vec = counts_vec + cnt * (lax.iota(jnp.int32, L) == b)

        nbig = cnt >> 11

        def fbig(j, _, b=b):
            pltpu.sync_copy(pbuf.at[pl.ds(j * 2048, 2048)],
                            slab_hbm.at[b, wid, pl.ds(j * 2048, 2048)])
            return 0

        lax.fori_loop(0, nbig, fbig, 0)

        def fsm(j, _, b=b):
            pltpu.sync_copy(pbuf.at[pl.ds(j * 128, 128)],
                            slab_hbm.at[b, wid, pl.ds(j * 128, 128)])
            return 0

        lax.fori_loop(nbig * L, cnt >> 7, fsm, 0)
    cnt_v[...] = counts_vec
    pltpu.sync_copy(cnt_v, counts_hbm.at[wid])


def _bin_kernel(mesh):
  return functools.partial(
    pl.kernel, _bin_body, mesh=mesh, compiler_params=_CP_NL,
    out_type=[jax.ShapeDtypeStruct((NB, NW, CAPW), jnp.int32),
              jax.ShapeDtypeStruct((NW, L), jnp.int32)],
    scratch_types=[pltpu.VMEM((SUB,), jnp.int32),
                   pltpu.VMEM((SUB,), jnp.int32),
                   pltpu.VMEM((PBUF,), jnp.int32),
                   pltpu.VMEM((L,), jnp.int32),
                   pltpu.SemaphoreType.DMA],
    cost_estimate=_SC_COST, name="gcn_bin")()


DGB = 2048


def _deg_body(slab_hbm, counts_hbm, deg_hbm, qbuf, dstl2, dtail_v, ones_v,
              zero_v, cvec_v, dacc, sem):
    c = lax.axis_index("c")
    s = lax.axis_index("s")

    @pl.loop(0, CH)
    def _(r):
        ones_v[r, pl.ds(0, L)] = jnp.full((L,), 1.0, jnp.float32)
        zero_v[r, pl.ds(0, L)] = jnp.zeros((L,), jnp.float32)

    for b in range(NB):
        @pl.when((b % NC) == c)
        def _(b=b):
            @pl.loop(0, BK // NS // 128)
            def _(j):
                pltpu.sync_copy(
                    zero_v, dacc.at[pl.ds(s * (BK // NS) + j * 128, 128)])
            plsc.subcore_barrier()
            for k in range(2):
                wseg = s * 2 + k
                pltpu.sync_copy(counts_hbm.at[wseg], cvec_v)
                n = cvec_v[...][b]

                def bulk(j, _, b=b, wseg=wseg):
                    pltpu.sync_copy(slab_hbm.at[b, wseg, pl.ds(j * DGB, DGB)],
                                    qbuf)

                    @pl.loop(0, DGB // CH)
                    def _(jj):
                        @pl.loop(0, CH // L)
                        def _(i):
                            dstl2[jj, pl.ds(i * L, L)] = (
                                lax.shift_right_logical(
                                    qbuf[pl.ds(jj * CH + i * L, L)], 17))

                    @pl.loop(0, DGB // CH)
                    def _(jj):
                        pltpu.sync_copy(ones_v, dacc.at[dstl2.at[jj]],
                                        add=True)
                    return 0

                nbig = n // DGB
                lax.fori_loop(0, nbig, bulk, 0)

                def tail(j, _, b=b, wseg=wseg):
                    pltpu.sync_copy(
                        slab_hbm.at[b, wseg, pl.ds(j * CH, CH)],
                        qbuf.at[pl.ds(0, CH)])

                    @pl.loop(0, CH // L)
                    def _(i):
                        dtail_v[pl.ds(i * L, L)] = lax.shift_right_logical(
                            qbuf[pl.ds(i * L, L)], 17)

                    pltpu.sync_copy(ones_v, dacc.at[dtail_v], add=True)
                    return 0

                lax.fori_loop(nbig * (DGB // CH), n // CH, tail, 0)
            plsc.subcore_barrier()
            pltpu.sync_copy(
                dacc.at[pl.ds(s * (BK // NS), BK // NS)],
                deg_hbm.at[pl.ds(b * BK + s * (BK // NS), BK // NS)])


def _deg_kernel(mesh):
  return functools.partial(
    pl.kernel, _deg_body, mesh=mesh, compiler_params=_CP,
    out_type=jax.ShapeDtypeStruct((NROWS, L), jnp.float32),
    scratch_types=[pltpu.VMEM((DGB,), jnp.int32),
                   pltpu.VMEM((DGB // CH, CH), jnp.int32),
                   pltpu.VMEM((CH,), jnp.int32),
                   pltpu.VMEM((CH, L), jnp.float32),
                   pltpu.VMEM((128, L), jnp.float32),
                   pltpu.VMEM((L,), jnp.int32),
                   pltpu.VMEM_SHARED((BK + L, L), jnp.float32),
                   pltpu.SemaphoreType.DMA],
    cost_estimate=_SC_COST, name="gcn_deg")()


def _acc_body(slab_hbm, counts_hbm, g_hbm, s_hbm, qb, s2, d2, rall,
              zero_v, cvec_v, acc, semg, sems):
    c = lax.axis_index("c")
    s = lax.axis_index("s")
    G = GACC

    @pl.loop(0, 128)
    def _(r):
        for j in range(HID // L):
            zero_v[r, pl.ds(j * L, L)] = jnp.zeros((L,), jnp.float32)

    def unpack_chunk(jj, base):
        @pl.loop(0, CH // L)
        def _(i):
            q = qb[pl.ds(base + i * L, L)]
            s2[jj, pl.ds(i * L, L)] = q & 0x1FFFF
            d2[jj, pl.ds(i * L, L)] = lax.shift_right_logical(q, 17)

    for b in range(NB):
        @pl.when((b % NC) == c)
        def _(b=b):
            @pl.loop(0, BK // NS // 128)
            def _(j):
                pltpu.sync_copy(
                    zero_v, acc.at[pl.ds(s * (BK // NS) + j * 128, 128)])
            plsc.subcore_barrier()
            for k in range(2):
                wseg = s * 2 + k
                pltpu.sync_copy(counts_hbm.at[wseg], cvec_v)
                n = cvec_v[...][b]

                def group(t, _, b=b, wseg=wseg):
                    pltpu.sync_copy(
                        slab_hbm.at[b, wseg, pl.ds(t * (G * CH), G * CH)], qb)
                    for jj in range(G):
                        unpack_chunk(jj, jj * CH)
                    gh = [pltpu.async_copy(g_hbm.at[s2.at[jj]],
                                           rall.at[pl.ds(jj * CH, CH)],
                                           semg.at[jj])
                          for jj in range(G)]
                    sh = []
                    for jj in range(G):
                        gh[jj].wait()
                        sh.append(pltpu.async_copy(
                            rall.at[pl.ds(jj * CH, CH)], acc.at[d2.at[jj]],
                            sems.at[jj], add=True))
                    for h in sh:
                        h.wait()
                    return 0

                ngr = n // (G * CH)
                lax.fori_loop(0, ngr, group, 0)

                def chunk(j, _, b=b, wseg=wseg):
                    pltpu.sync_copy(slab_hbm.at[b, wseg, pl.ds(j * CH, CH)],
                                    qb.at[pl.ds(0, CH)])
                    unpack_chunk(0, 0)
                    pltpu.async_copy(g_hbm.at[s2.at[0]],
                                     rall.at[pl.ds(0, CH)], semg.at[0]).wait()
                    pltpu.sync_copy(rall.at[pl.ds(0, CH)], acc.at[d2.at[0]],
                                    add=True)
                    return 0

                lax.fori_loop(ngr * G, n // CH, chunk, 0)
            plsc.subcore_barrier()
            pltpu.sync_copy(
                acc.at[pl.ds(s * (BK // NS), BK // NS)],
                s_hbm.at[pl.ds(b * BK + s * (BK // NS), BK // NS)])


def _acc_kernel(mesh):
  return functools.partial(
    pl.kernel, _acc_body, mesh=mesh, compiler_params=_CP,
    out_type=jax.ShapeDtypeStruct((NROWS, HID), jnp.float32),
    scratch_types=[pltpu.VMEM((GACC * CH,), jnp.int32),
                   pltpu.VMEM((GACC, CH), jnp.int32),
                   pltpu.VMEM((GACC, CH), jnp.int32),
                   pltpu.VMEM((GACC * CH, HID), jnp.float32),
                   pltpu.VMEM((128, HID), jnp.float32),
                   pltpu.VMEM((L,), jnp.int32),
                   pltpu.VMEM_SHARED((BK + L, HID), jnp.float32),
                   pltpu.SemaphoreType.DMA((GACC,)),
                   pltpu.SemaphoreType.DMA((GACC,))],
    cost_estimate=_SC_COST, name="gcn_acc")()


def _t1_body(x_ref, deg_ref, w_ref, out_ref):
    dinv = lax.rsqrt(deg_ref[:, 0:1] + 1.0)
    xb = x_ref[...]
    w = w_ref[...]
    h = (xb[:, 0:1] * w[0:1, :] + xb[:, 1:2] * w[1:2, :]
         + xb[:, 2:3] * w[2:3, :])
    out_ref[...] = dinv * h


def _tmid_body(s_ref, g_ref, deg_ref, w_ref, b_ref, out_ref):
    dinv = lax.rsqrt(deg_ref[:, 0:1] + 1.0)
    t = jnp.maximum(dinv * (s_ref[...] + g_ref[...]) + b_ref[0:1, :], 0.0)
    out_ref[...] = dinv * jnp.dot(t, w_ref[...],
                                  preferred_element_type=jnp.float32)


def _t4_body(s_ref, g_ref, deg_ref, b3_ref, m1_ref, mb1_ref, m2_ref, mb2_ref,
             out_ref):
    dinv = lax.rsqrt(deg_ref[:, 0:1] + 1.0)
    h3 = jnp.maximum(dinv * (s_ref[...] + g_ref[...]) + b3_ref[0:1, :], 0.0)
    h4 = jnp.maximum(jnp.dot(h3, m1_ref[...],
                             preferred_element_type=jnp.float32)
                     + mb1_ref[0:1, :], 0.0)
    z = jnp.sum(h4 * m2_ref[0:1, :], axis=1, keepdims=True) + mb2_ref[0:1, 0:1]
    out_ref[...] = jax.nn.sigmoid(z)


def _row_spec(cols):
    return pl.BlockSpec((RB, cols), lambda i: (i, 0))


def _full_spec(r, cols):
    return pl.BlockSpec((r, cols), lambda i: (0, 0))


_t1_call = pl.pallas_call(
    _t1_body, grid=(GRID,),
    in_specs=[_row_spec(3), _row_spec(L), _full_spec(8, HID)],
    out_specs=_row_spec(HID),
    out_shape=jax.ShapeDtypeStruct((NPAD, HID), jnp.float32))

_tmid_call = pl.pallas_call(
    _tmid_body, grid=(GRID,),
    in_specs=[_row_spec(HID), _row_spec(HID), _row_spec(L),
              _full_spec(HID, HID), _full_spec(8, HID)],
    out_specs=_row_spec(HID),
    out_shape=jax.ShapeDtypeStruct((NPAD, HID), jnp.float32))

_t4_call = pl.pallas_call(
    _t4_body, grid=(GRID,),
    in_specs=[_row_spec(HID), _row_spec(HID), _row_spec(L),
              _full_spec(8, HID), _full_spec(HID, HID), _full_spec(8, HID),
              _full_spec(8, HID), _full_spec(8, HID)],
    out_specs=pl.BlockSpec((RB, 1), lambda i: (i, 0)),
    out_shape=jax.ShapeDtypeStruct((N, 1), jnp.float32))


@functools.lru_cache(maxsize=1)
def _sc_kernels():
    mesh = plsc.VectorSubcoreMesh(core_axis_name="c", subcore_axis_name="s",
                                  num_cores=NC, num_subcores=NS)
    return _bin_kernel(mesh), _deg_kernel(mesh), _acc_kernel(mesh)


def kernel(x, edge_index, W1, b1, W2, b2, W3, b3, M1, mb1, M2, mb2):
    _bin_call, _deg_call, _acc_call = _sc_kernels()
    slab, counts = _bin_call(edge_index)
    deg = _deg_call(slab, counts)

    w1p = jnp.pad(W1, ((0, 5), (0, 0)))
    b1p = jnp.broadcast_to(b1.reshape(1, HID), (8, HID))
    b2p = jnp.broadcast_to(b2.reshape(1, HID), (8, HID))
    b3p = jnp.broadcast_to(b3.reshape(1, HID), (8, HID))
    mb1p = jnp.broadcast_to(mb1.reshape(1, HID), (8, HID))
    m2p = jnp.broadcast_to(M2.reshape(1, HID), (8, HID))
    mb2p = jnp.broadcast_to(mb2.reshape(1, 1), (8, HID))

    g1 = _t1_call(x, deg, w1p)
    s1 = _acc_call(slab, counts, g1)
    g2 = _tmid_call(s1, g1, deg, W2, b1p)
    s2 = _acc_call(slab, counts, g2)
    g3 = _tmid_call(s2, g2, deg, W3, b2p)
    s3 = _acc_call(slab, counts, g3)
    return _t4_call(s3, g3, deg, b3p, M1, mb1p, m2p, mb2p)


# speedup vs baseline: 1.7461x; 1.0208x over previous
import functools

import jax
import jax.numpy as jnp
import numpy as np
from jax import lax
from jax.experimental import pallas as pl
from jax.experimental.pallas import tpu as pltpu
from jax.experimental.pallas import tpu_sc as plsc

N = 100000
E = 1600000
HID = 64

NC = 2
NS = 16
NW = NC * NS
L = 16

_CP = pltpu.CompilerParams(use_tc_tiling_on_sc=False)
_CP_NL = pltpu.CompilerParams(use_tc_tiling_on_sc=False,
                              needs_layout_passes=False)

BK = 1 << 14
NB = (N + BK - 1) // BK
NROWS = NB * BK
DUMP = BK

E_PER_W = E // NW
SUB = 10000
NSUB = E_PER_W // SUB
PBUF = 50688
CAPW = 50688
CH = 128
GACC = 6
SEG_ALIGN = 128

PADQ = np.int32(np.uint32(N | (np.uint32(DUMP) << 17)))

NPAD = 100352
RB = 2048
GRID = NPAD // RB

_SC_COST = pl.CostEstimate(flops=0, transcendentals=0,
                           bytes_accessed=E * 4)


def _bin_body(ei_hbm, slab_hbm, counts_hbm, src_v, dst_v, pbuf, cnt_v, sem):
    c = lax.axis_index("c")
    s = lax.axis_index("s")
    wid = s * NC + c
    base = wid * E_PER_W
    counts_vec = jnp.zeros((L,), jnp.int32)
    padv = jnp.full((L,), PADQ, jnp.int32)
    for b in range(NB):
        def subchunk(k, cnt, b=b):
            off = base + k * SUB
            pltpu.sync_copy(ei_hbm.at[0, pl.ds(off, SUB)], src_v)
            pltpu.sync_copy(ei_hbm.at[1, pl.ds(off, SUB)], dst_v)

            def vec(i, cnt):
                sv = src_v[pl.ds(i * L, L)]
                dv = dst_v[pl.ds(i * L, L)]
                m = (dv >> 14) == b
                q = sv | ((dv & (BK - 1)) << 17)
                plsc.store_compressed(pbuf.at[pl.ds(cnt, L)], q, mask=m)
                return cnt + jnp.sum(m.astype(jnp.int32))

            return lax.fori_loop(0, SUB // L, vec, cnt)

        cnt = lax.fori_loop(0, NSUB, subchunk, jnp.int32(0))
        pbuf[pl.ds(cnt, L)] = padv
        cnt = (cnt + 15) & jnp.int32(-16)

        def wbody(cc):
            pbuf[pl.ds(cc, L)] = padv
            return cc + L

        cnt = lax.while_loop(lambda cc: (cc & 127) != 0, wbody, cnt)
        counts_vec = counts_vec + cnt * (lax.iota(jnp.int32, L) == b)

        nbig = cnt >> 11

        def fbig(j, _, b=b):
            pltpu.sync_copy(pbuf.at[pl.ds(j * 2048, 2048)],
                            slab_hbm.at[b, wid, pl.ds(j * 2048, 2048)])
            return 0

        lax.fori_loop(0, nbig, fbig, 0)

        def fsm(j, _, b=b):
            pltpu.sync_copy(pbuf.at[pl.ds(j * 128, 128)],
                            slab_hbm.at[b, wid, pl.ds(j * 128, 128)])
            return 0

        lax.fori_loop(nbig * L, cnt >> 7, fsm, 0)
    cnt_v[...] = counts_vec
    pltpu.sync_copy(cnt_v, counts_hbm.at[wid])


def _bin_kernel(mesh):
  return functools.partial(
    pl.kernel, _bin_body, mesh=mesh, compiler_params=_CP_NL,
    out_type=[jax.ShapeDtypeStruct((NB, NW, CAPW), jnp.int32),
              jax.ShapeDtypeStruct((NW, L), jnp.int32)],
    scratch_types=[pltpu.VMEM((SUB,), jnp.int32),
                   pltpu.VMEM((SUB,), jnp.int32),
                   pltpu.VMEM((PBUF,), jnp.int32),
                   pltpu.VMEM((L,), jnp.int32),
                   pltpu.SemaphoreType.DMA],
    cost_estimate=_SC_COST, name="gcn_bin")()


DGB = 2048


def _deg_body(slab_hbm, counts_hbm, deg_hbm, qbuf, dstl2, dtail_v, ones_v,
              zero_v, cvec_v, dacc, sem):
    c = lax.axis_index("c")
    s = lax.axis_index("s")

    @pl.loop(0, CH)
    def _(r):
        ones_v[r, pl.ds(0, L)] = jnp.full((L,), 1.0, jnp.float32)
        zero_v[r, pl.ds(0, L)] = jnp.zeros((L,), jnp.float32)

    for b in range(NB):
        @pl.when((b % NC) == c)
        def _(b=b):
            @pl.loop(0, BK // NS // 128)
            def _(j):
                pltpu.sync_copy(
                    zero_v, dacc.at[pl.ds(s * (BK // NS) + j * 128, 128)])
            plsc.subcore_barrier()
            for k in range(2):
                wseg = s * 2 + k
                pltpu.sync_copy(counts_hbm.at[wseg], cvec_v)
                n = cvec_v[...][b]

                def bulk(j, _, b=b, wseg=wseg):
                    pltpu.sync_copy(slab_hbm.at[b, wseg, pl.ds(j * DGB, DGB)],
                                    qbuf)

                    @pl.loop(0, DGB // CH)
                    def _(jj):
                        @pl.loop(0, CH // L)
                        def _(i):
                            dstl2[jj, pl.ds(i * L, L)] = (
                                lax.shift_right_logical(
                                    qbuf[pl.ds(jj * CH + i * L, L)], 17))

                    @pl.loop(0, DGB // CH)
                    def _(jj):
                        pltpu.sync_copy(ones_v, dacc.at[dstl2.at[jj]],
                                        add=True)
                    return 0

                nbig = n // DGB
                lax.fori_loop(0, nbig, bulk, 0)

                def tail(j, _, b=b, wseg=wseg):
                    pltpu.sync_copy(
                        slab_hbm.at[b, wseg, pl.ds(j * CH, CH)],
                        qbuf.at[pl.ds(0, CH)])

                    @pl.loop(0, CH // L)
                    def _(i):
                        dtail_v[pl.ds(i * L, L)] = lax.shift_right_logical(
                            qbuf[pl.ds(i * L, L)], 17)

                    pltpu.sync_copy(ones_v, dacc.at[dtail_v], add=True)
                    return 0

                lax.fori_loop(nbig * (DGB // CH), n // CH, tail, 0)
            plsc.subcore_barrier()
            pltpu.sync_copy(
                dacc.at[pl.ds(s * (BK // NS), BK // NS)],
                deg_hbm.at[pl.ds(b * BK + s * (BK // NS), BK // NS)])


def _deg_kernel(mesh):
  return functools.partial(
    pl.kernel, _deg_body, mesh=mesh, compiler_params=_CP,
    out_type=jax.ShapeDtypeStruct((NROWS, L), jnp.float32),
    scratch_types=[pltpu.VMEM((DGB,), jnp.int32),
                   pltpu.VMEM((DGB // CH, CH), jnp.int32),
                   pltpu.VMEM((CH,), jnp.int32),
                   pltpu.VMEM((CH, L), jnp.float32),
                   pltpu.VMEM((128, L), jnp.float32),
                   pltpu.VMEM((L,), jnp.int32),
                   pltpu.VMEM_SHARED((BK + L, L), jnp.float32),
                   pltpu.SemaphoreType.DMA],
    cost_estimate=_SC_COST, name="gcn_deg")()


def _acc_body(slab_hbm, counts_hbm, g_hbm, s_hbm, qb, s2, d2, rall,
              zero_v, cvec_v, acc, semg, sems, semq):
    c = lax.axis_index("c")
    s = lax.axis_index("s")
    G = GACC

    @pl.loop(0, 128)
    def _(r):
        for j in range(HID // L):
            zero_v[r, pl.ds(j * L, L)] = jnp.zeros((L,), jnp.float32)

    def unpack_chunk(h, jj, base):
        @pl.loop(0, CH // L)
        def _(i):
            q = qb[h, pl.ds(base + i * L, L)]
            s2[jj, pl.ds(i * L, L)] = q & 0x1FFFF
            d2[jj, pl.ds(i * L, L)] = lax.shift_right_logical(q, 17)

    for b in range(NB):
        @pl.when((b % NC) == c)
        def _(b=b):
            @pl.loop(0, BK // NS // 128)
            def _(j):
                pltpu.sync_copy(
                    zero_v, acc.at[pl.ds(s * (BK // NS) + j * 128, 128)])
            plsc.subcore_barrier()
            for k in range(2):
                wseg = s * 2 + k
                pltpu.sync_copy(counts_hbm.at[wseg], cvec_v)
                n = cvec_v[...][b]

                ngr0 = n // (G * CH)

                def group(t, _, b=b, wseg=wseg, ngr=None):
                    h = t & 1

                    @pl.when(t == 0)
                    def _():
                        pltpu.sync_copy(
                            slab_hbm.at[b, wseg, pl.ds(0, G * CH)],
                            qb.at[0])

                    @pl.when(t > 0)
                    def _():
                        pltpu.make_async_copy(
                            slab_hbm.at[b, wseg,
                                        pl.ds(t * (G * CH), G * CH)],
                            qb.at[h], semq.at[h]).wait()

                    @pl.when(t + 1 < ngr0)
                    def _():
                        pltpu.async_copy(
                            slab_hbm.at[b, wseg,
                                        pl.ds((t + 1) * (G * CH), G * CH)],
                            qb.at[1 - h], semq.at[1 - h])
                    for jj in range(G):
                        unpack_chunk(h, jj, jj * CH)
                    gh = [pltpu.async_copy(g_hbm.at[s2.at[jj]],
                                           rall.at[pl.ds(jj * CH, CH)],
                                           semg.at[jj])
                          for jj in range(G)]
                    sh = []
                    for jj in range(G):
                        gh[jj].wait()
                        sh.append(pltpu.async_copy(
                            rall.at[pl.ds(jj * CH, CH)], acc.at[d2.at[jj]],
                            sems.at[jj], add=True))
                    for h in sh:
                        h.wait()
                    return 0

                ngr = n // (G * CH)
                lax.fori_loop(0, ngr, group, 0)

                def chunk(j, _, b=b, wseg=wseg):
                    pltpu.sync_copy(slab_hbm.at[b, wseg, pl.ds(j * CH, CH)],
                                    qb.at[0, pl.ds(0, CH)])
                    unpack_chunk(0, 0, 0)
                    pltpu.async_copy(g_hbm.at[s2.at[0]],
                                     rall.at[pl.ds(0, CH)], semg.at[0]).wait()
                    pltpu.sync_copy(rall.at[pl.ds(0, CH)], acc.at[d2.at[0]],
                                    add=True)
                    return 0

                lax.fori_loop(ngr * G, n // CH, chunk, 0)
            plsc.subcore_barrier()
            pltpu.sync_copy(
                acc.at[pl.ds(s * (BK // NS), BK // NS)],
                s_hbm.at[pl.ds(b * BK + s * (BK // NS), BK // NS)])


def _acc_kernel(mesh):
  return functools.partial(
    pl.kernel, _acc_body, mesh=mesh, compiler_params=_CP,
    out_type=jax.ShapeDtypeStruct((NROWS, HID), jnp.float32),
    scratch_types=[pltpu.VMEM((2, GACC * CH), jnp.int32),
                   pltpu.VMEM((GACC, CH), jnp.int32),
                   pltpu.VMEM((GACC, CH), jnp.int32),
                   pltpu.VMEM((GACC * CH, HID), jnp.float32),
                   pltpu.VMEM((128, HID), jnp.float32),
                   pltpu.VMEM((L,), jnp.int32),
                   pltpu.VMEM_SHARED((BK + L, HID), jnp.float32),
                   pltpu.SemaphoreType.DMA((GACC,)),
                   pltpu.SemaphoreType.DMA((GACC,)),
                   pltpu.SemaphoreType.DMA((2,))],
    cost_estimate=_SC_COST, name="gcn_acc")()


def _t1_body(x_ref, deg_ref, w_ref, out_ref):
    dinv = lax.rsqrt(deg_ref[:, 0:1] + 1.0)
    xb = x_ref[...]
    w = w_ref[...]
    h = (xb[:, 0:1] * w[0:1, :] + xb[:, 1:2] * w[1:2, :]
         + xb[:, 2:3] * w[2:3, :])
    out_ref[...] = dinv * h


def _tmid_body(s_ref, g_ref, deg_ref, w_ref, b_ref, out_ref):
    dinv = lax.rsqrt(deg_ref[:, 0:1] + 1.0)
    t = jnp.maximum(dinv * (s_ref[...] + g_ref[...]) + b_ref[0:1, :], 0.0)
    out_ref[...] = dinv * jnp.dot(t, w_ref[...],
                                  preferred_element_type=jnp.float32)


def _t4_body(s_ref, g_ref, deg_ref, b3_ref, m1_ref, mb1_ref, m2_ref, mb2_ref,
             out_ref):
    dinv = lax.rsqrt(deg_ref[:, 0:1] + 1.0)
    h3 = jnp.maximum(dinv * (s_ref[...] + g_ref[...]) + b3_ref[0:1, :], 0.0)
    h4 = jnp.maximum(jnp.dot(h3, m1_ref[...],
                             preferred_element_type=jnp.float32)
                     + mb1_ref[0:1, :], 0.0)
    z = jnp.sum(h4 * m2_ref[0:1, :], axis=1, keepdims=True) + mb2_ref[0:1, 0:1]
    out_ref[...] = jax.nn.sigmoid(z)


def _row_spec(cols):
    return pl.BlockSpec((RB, cols), lambda i: (i, 0))


def _full_spec(r, cols):
    return pl.BlockSpec((r, cols), lambda i: (0, 0))


_t1_call = pl.pallas_call(
    _t1_body, grid=(GRID,),
    in_specs=[_row_spec(3), _row_spec(L), _full_spec(8, HID)],
    out_specs=_row_spec(HID),
    out_shape=jax.ShapeDtypeStruct((NPAD, HID), jnp.float32))

_tmid_call = pl.pallas_call(
    _tmid_body, grid=(GRID,),
    in_specs=[_row_spec(HID), _row_spec(HID), _row_spec(L),
              _full_spec(HID, HID), _full_spec(8, HID)],
    out_specs=_row_spec(HID),
    out_shape=jax.ShapeDtypeStruct((NPAD, HID), jnp.float32))

_t4_call = pl.pallas_call(
    _t4_body, grid=(GRID,),
    in_specs=[_row_spec(HID), _row_spec(HID), _row_spec(L),
              _full_spec(8, HID), _full_spec(HID, HID), _full_spec(8, HID),
              _full_spec(8, HID), _full_spec(8, HID)],
    out_specs=pl.BlockSpec((RB, 1), lambda i: (i, 0)),
    out_shape=jax.ShapeDtypeStruct((N, 1), jnp.float32))


@functools.lru_cache(maxsize=1)
def _sc_kernels():
    mesh = plsc.VectorSubcoreMesh(core_axis_name="c", subcore_axis_name="s",
                                  num_cores=NC, num_subcores=NS)
    return _bin_kernel(mesh), _deg_kernel(mesh), _acc_kernel(mesh)


def kernel(x, edge_index, W1, b1, W2, b2, W3, b3, M1, mb1, M2, mb2):
    _bin_call, _deg_call, _acc_call = _sc_kernels()
    slab, counts = _bin_call(edge_index)
    deg = _deg_call(slab, counts)

    w1p = jnp.pad(W1, ((0, 5), (0, 0)))
    b1p = jnp.broadcast_to(b1.reshape(1, HID), (8, HID))
    b2p = jnp.broadcast_to(b2.reshape(1, HID), (8, HID))
    b3p = jnp.broadcast_to(b3.reshape(1, HID), (8, HID))
    mb1p = jnp.broadcast_to(mb1.reshape(1, HID), (8, HID))
    m2p = jnp.broadcast_to(M2.reshape(1, HID), (8, HID))
    mb2p = jnp.broadcast_to(mb2.reshape(1, 1), (8, HID))

    g1 = _t1_call(x, deg, w1p)
    s1 = _acc_call(slab, counts, g1)
    g2 = _tmid_call(s1, g1, deg, W2, b1p)
    s2 = _acc_call(slab, counts, g2)
    g3 = _tmid_call(s2, g2, deg, W3, b2p)
    s3 = _acc_call(slab, counts, g3)
    return _t4_call(s3, g3, deg, b3p, M1, mb1p, m2p, mb2p)


# speedup vs baseline: 1.7807x; 1.0198x over previous
import functools

import jax
import jax.numpy as jnp
import numpy as np
from jax import lax
from jax.experimental import pallas as pl
from jax.experimental.pallas import tpu as pltpu
from jax.experimental.pallas import tpu_sc as plsc

N = 100000
E = 1600000
HID = 64

NC = 2
NS = 16
NW = NC * NS
L = 16

_CP = pltpu.CompilerParams(use_tc_tiling_on_sc=False)
_CP_NL = pltpu.CompilerParams(use_tc_tiling_on_sc=False,
                              needs_layout_passes=False)

BK = 1 << 14
NB = (N + BK - 1) // BK
NROWS = NB * BK
DUMP = BK

E_PER_W = E // NW
SUB = 10000
NSUB = E_PER_W // SUB
PBUF = 50688
CAPW = 50688
CH = 128
GACC = 6
SEG_ALIGN = 128

PADQ = np.int32(np.uint32(N | (np.uint32(DUMP) << 17)))

NPAD = 100352
RB = 2048
GRID = NPAD // RB

_SC_COST = pl.CostEstimate(flops=0, transcendentals=0,
                           bytes_accessed=E * 4)


def _bin_body(ei_hbm, slab_hbm, counts_hbm, src_v, dst_v, pbuf, cnt_v, sem,
              semp):
    c = lax.axis_index("c")
    s = lax.axis_index("s")
    wid = s * NC + c
    base = wid * E_PER_W
    counts_vec = jnp.zeros((L,), jnp.int32)
    padv = jnp.full((L,), PADQ, jnp.int32)
    for b in range(NB):
        def subchunk(k, cnt, b=b):
            off = base + k * SUB
            h = k & 1

            @pl.when(k == 0)
            def _():
                pltpu.sync_copy(ei_hbm.at[0, pl.ds(off, SUB)], src_v.at[0])
                pltpu.sync_copy(ei_hbm.at[1, pl.ds(off, SUB)], dst_v.at[0])

            @pl.when(k > 0)
            def _():
                pltpu.make_async_copy(ei_hbm.at[0, pl.ds(off, SUB)],
                                      src_v.at[h], semp.at[h]).wait()
                pltpu.make_async_copy(ei_hbm.at[1, pl.ds(off, SUB)],
                                      dst_v.at[h], semp.at[h]).wait()

            @pl.when(k + 1 < NSUB)
            def _():
                noff = base + (k + 1) * SUB
                pltpu.async_copy(ei_hbm.at[0, pl.ds(noff, SUB)],
                                 src_v.at[1 - h], semp.at[1 - h])
                pltpu.async_copy(ei_hbm.at[1, pl.ds(noff, SUB)],
                                 dst_v.at[1 - h], semp.at[1 - h])

            def vec(i, cnt):
                sv = src_v[h, pl.ds(i * L, L)]
                dv = dst_v[h, pl.ds(i * L, L)]
                m = (dv >> 14) == b
                q = sv | ((dv & (BK - 1)) << 17)
                plsc.store_compressed(pbuf.at[pl.ds(cnt, L)], q, mask=m)
                return cnt + jnp.sum(m.astype(jnp.int32))

            return lax.fori_loop(0, SUB // L, vec, cnt)

        cnt = lax.fori_loop(0, NSUB, subchunk, jnp.int32(0))
        pbuf[pl.ds(cnt, L)] = padv
        cnt = (cnt + 15) & jnp.int32(-16)

        def wbody(cc):
            pbuf[pl.ds(cc, L)] = padv
            return cc + L

        cnt = lax.while_loop(lambda cc: (cc & 127) != 0, wbody, cnt)
        counts_vec = counts_vec + cnt * (lax.iota(jnp.int32, L) == b)

        nbig = cnt >> 11

        def fbig(j, _, b=b):
            pltpu.sync_copy(pbuf.at[pl.ds(j * 2048, 2048)],
                            slab_hbm.at[b, wid, pl.ds(j * 2048, 2048)])
            return 0

        lax.fori_loop(0, nbig, fbig, 0)

        def fsm(j, _, b=b):
            pltpu.sync_copy(pbuf.at[pl.ds(j * 128, 128)],
                            slab_hbm.at[b, wid, pl.ds(j * 128, 128)])
            return 0

        lax.fori_loop(nbig * L, cnt >> 7, fsm, 0)
    cnt_v[...] = counts_vec
    pltpu.sync_copy(cnt_v, counts_hbm.at[wid])


def _bin_kernel(mesh):
  return functools.partial(
    pl.kernel, _bin_body, mesh=mesh, compiler_params=_CP_NL,
    out_type=[jax.ShapeDtypeStruct((NB, NW, CAPW), jnp.int32),
              jax.ShapeDtypeStruct((NW, L), jnp.int32)],
    scratch_types=[pltpu.VMEM((2, SUB), jnp.int32),
                   pltpu.VMEM((2, SUB), jnp.int32),
                   pltpu.VMEM((PBUF,), jnp.int32),
                   pltpu.VMEM((L,), jnp.int32),
                   pltpu.SemaphoreType.DMA,
                   pltpu.SemaphoreType.DMA((2,))],
    cost_estimate=_SC_COST, name="gcn_bin")()


DGB = 2048


def _deg_body(slab_hbm, counts_hbm, deg_hbm, qbuf, dstl2, dtail_v, ones_v,
              zero_v, cvec_v, dacc, sem):
    c = lax.axis_index("c")
    s = lax.axis_index("s")

    @pl.loop(0, CH)
    def _(r):
        ones_v[r, pl.ds(0, L)] = jnp.full((L,), 1.0, jnp.float32)
        zero_v[r, pl.ds(0, L)] = jnp.zeros((L,), jnp.float32)

    for b in range(NB):
        @pl.when((b % NC) == c)
        def _(b=b):
            @pl.loop(0, BK // NS // 128)
            def _(j):
                pltpu.sync_copy(
                    zero_v, dacc.at[pl.ds(s * (BK // NS) + j * 128, 128)])
            plsc.subcore_barrier()
            for k in range(2):
                wseg = s * 2 + k
                pltpu.sync_copy(counts_hbm.at[wseg], cvec_v)
                n = cvec_v[...][b]

                def bulk(j, _, b=b, wseg=wseg):
                    pltpu.sync_copy(slab_hbm.at[b, wseg, pl.ds(j * DGB, DGB)],
                                    qbuf)

                    @pl.loop(0, DGB // CH)
                    def _(jj):
                        @pl.loop(0, CH // L)
                        def _(i):
                            dstl2[jj, pl.ds(i * L, L)] = (
                                lax.shift_right_logical(
                                    qbuf[pl.ds(jj * CH + i * L, L)], 17))

                    @pl.loop(0, DGB // CH)
                    def _(jj):
                        pltpu.sync_copy(ones_v, dacc.at[dstl2.at[jj]],
                                        add=True)
                    return 0

                nbig = n // DGB
                lax.fori_loop(0, nbig, bulk, 0)

                def tail(j, _, b=b, wseg=wseg):
                    pltpu.sync_copy(
                        slab_hbm.at[b, wseg, pl.ds(j * CH, CH)],
                        qbuf.at[pl.ds(0, CH)])

                    @pl.loop(0, CH // L)
                    def _(i):
                        dtail_v[pl.ds(i * L, L)] = lax.shift_right_logical(
                            qbuf[pl.ds(i * L, L)], 17)

                    pltpu.sync_copy(ones_v, dacc.at[dtail_v], add=True)
                    return 0

                lax.fori_loop(nbig * (DGB // CH), n // CH, tail, 0)
            plsc.subcore_barrier()
            pltpu.sync_copy(
                dacc.at[pl.ds(s * (BK // NS), BK // NS)],
                deg_hbm.at[pl.ds(b * BK + s * (BK // NS), BK // NS)])


def _deg_kernel(mesh):
  return functools.partial(
    pl.kernel, _deg_body, mesh=mesh, compiler_params=_CP,
    out_type=jax.ShapeDtypeStruct((NROWS, L), jnp.float32),
    scratch_types=[pltpu.VMEM((DGB,), jnp.int32),
                   pltpu.VMEM((DGB // CH, CH), jnp.int32),
                   pltpu.VMEM((CH,), jnp.int32),
                   pltpu.VMEM((CH, L), jnp.float32),
                   pltpu.VMEM((128, L), jnp.float32),
                   pltpu.VMEM((L,), jnp.int32),
                   pltpu.VMEM_SHARED((BK + L, L), jnp.float32),
                   pltpu.SemaphoreType.DMA],
    cost_estimate=_SC_COST, name="gcn_deg")()


def _acc_body(slab_hbm, counts_hbm, g_hbm, s_hbm, qb, s2, d2, rall,
              zero_v, cvec_v, acc, semg, sems, semq):
    c = lax.axis_index("c")
    s = lax.axis_index("s")
    G = GACC

    @pl.loop(0, 128)
    def _(r):
        for j in range(HID // L):
            zero_v[r, pl.ds(j * L, L)] = jnp.zeros((L,), jnp.float32)

    def unpack_chunk(h, jj, base):
        @pl.loop(0, CH // L)
        def _(i):
            q = qb[h, pl.ds(base + i * L, L)]
            s2[jj, pl.ds(i * L, L)] = q & 0x1FFFF
            d2[jj, pl.ds(i * L, L)] = lax.shift_right_logical(q, 17)

    for b in range(NB):
        @pl.when((b % NC) == c)
        def _(b=b):
            @pl.loop(0, BK // NS // 128)
            def _(j):
                pltpu.sync_copy(
                    zero_v, acc.at[pl.ds(s * (BK // NS) + j * 128, 128)])
            plsc.subcore_barrier()
            for k in range(2):
                wseg = s * 2 + k
                pltpu.sync_copy(counts_hbm.at[wseg], cvec_v)
                n = cvec_v[...][b]

                ngr0 = n // (G * CH)

                def group(t, _, b=b, wseg=wseg, ngr=None):
                    h = t & 1

                    @pl.when(t == 0)
                    def _():
                        pltpu.sync_copy(
                            slab_hbm.at[b, wseg, pl.ds(0, G * CH)],
                            qb.at[0])

                    @pl.when(t > 0)
                    def _():
                        pltpu.make_async_copy(
                            slab_hbm.at[b, wseg,
                                        pl.ds(t * (G * CH), G * CH)],
                            qb.at[h], semq.at[h]).wait()

                    @pl.when(t + 1 < ngr0)
                    def _():
                        pltpu.async_copy(
                            slab_hbm.at[b, wseg,
                                        pl.ds((t + 1) * (G * CH), G * CH)],
                            qb.at[1 - h], semq.at[1 - h])
                    for jj in range(G):
                        unpack_chunk(h, jj, jj * CH)
                    gh = [pltpu.async_copy(g_hbm.at[s2.at[jj]],
                                           rall.at[pl.ds(jj * CH, CH)],
                                           semg.at[jj])
                          for jj in range(G)]
                    sh = []
                    for jj in range(G):
                        gh[jj].wait()
                        sh.append(pltpu.async_copy(
                            rall.at[pl.ds(jj * CH, CH)], acc.at[d2.at[jj]],
                            sems.at[jj], add=True))
                    for h in sh:
                        h.wait()
                    return 0

                ngr = n // (G * CH)
                lax.fori_loop(0, ngr, group, 0)

                def chunk(j, _, b=b, wseg=wseg):
                    pltpu.sync_copy(slab_hbm.at[b, wseg, pl.ds(j * CH, CH)],
                                    qb.at[0, pl.ds(0, CH)])
                    unpack_chunk(0, 0, 0)
                    pltpu.async_copy(g_hbm.at[s2.at[0]],
                                     rall.at[pl.ds(0, CH)], semg.at[0]).wait()
                    pltpu.sync_copy(rall.at[pl.ds(0, CH)], acc.at[d2.at[0]],
                                    add=True)
                    return 0

                lax.fori_loop(ngr * G, n // CH, chunk, 0)
            plsc.subcore_barrier()
            pltpu.sync_copy(
                acc.at[pl.ds(s * (BK // NS), BK // NS)],
                s_hbm.at[pl.ds(b * BK + s * (BK // NS), BK // NS)])


def _acc_kernel(mesh):
  return functools.partial(
    pl.kernel, _acc_body, mesh=mesh, compiler_params=_CP,
    out_type=jax.ShapeDtypeStruct((NROWS, HID), jnp.float32),
    scratch_types=[pltpu.VMEM((2, GACC * CH), jnp.int32),
                   pltpu.VMEM((GACC, CH), jnp.int32),
                   pltpu.VMEM((GACC, CH), jnp.int32),
                   pltpu.VMEM((GACC * CH, HID), jnp.float32),
                   pltpu.VMEM((128, HID), jnp.float32),
                   pltpu.VMEM((L,), jnp.int32),
                   pltpu.VMEM_SHARED((BK + L, HID), jnp.float32),
                   pltpu.SemaphoreType.DMA((GACC,)),
                   pltpu.SemaphoreType.DMA((GACC,)),
                   pltpu.SemaphoreType.DMA((2,))],
    cost_estimate=_SC_COST, name="gcn_acc")()


def _t1_body(x_ref, deg_ref, w_ref, out_ref):
    dinv = lax.rsqrt(deg_ref[:, 0:1] + 1.0)
    xb = x_ref[...]
    w = w_ref[...]
    h = (xb[:, 0:1] * w[0:1, :] + xb[:, 1:2] * w[1:2, :]
         + xb[:, 2:3] * w[2:3, :])
    out_ref[...] = dinv * h


def _tmid_body(s_ref, g_ref, deg_ref, w_ref, b_ref, out_ref):
    dinv = lax.rsqrt(deg_ref[:, 0:1] + 1.0)
    t = jnp.maximum(dinv * (s_ref[...] + g_ref[...]) + b_ref[0:1, :], 0.0)
    out_ref[...] = dinv * jnp.dot(t, w_ref[...],
                                  preferred_element_type=jnp.float32)


def _t4_body(s_ref, g_ref, deg_ref, b3_ref, m1_ref, mb1_ref, m2_ref, mb2_ref,
             out_ref):
    dinv = lax.rsqrt(deg_ref[:, 0:1] + 1.0)
    h3 = jnp.maximum(dinv * (s_ref[...] + g_ref[...]) + b3_ref[0:1, :], 0.0)
    h4 = jnp.maximum(jnp.dot(h3, m1_ref[...],
                             preferred_element_type=jnp.float32)
                     + mb1_ref[0:1, :], 0.0)
    z = jnp.sum(h4 * m2_ref[0:1, :], axis=1, keepdims=True) + mb2_ref[0:1, 0:1]
    out_ref[...] = jax.nn.sigmoid(z)


def _row_spec(cols):
    return pl.BlockSpec((RB, cols), lambda i: (i, 0))


def _full_spec(r, cols):
    return pl.BlockSpec((r, cols), lambda i: (0, 0))


_t1_call = pl.pallas_call(
    _t1_body, grid=(GRID,),
    in_specs=[_row_spec(3), _row_spec(L), _full_spec(8, HID)],
    out_specs=_row_spec(HID),
    out_shape=jax.ShapeDtypeStruct((NPAD, HID), jnp.float32))

_tmid_call = pl.pallas_call(
    _tmid_body, grid=(GRID,),
    in_specs=[_row_spec(HID), _row_spec(HID), _row_spec(L),
              _full_spec(HID, HID), _full_spec(8, HID)],
    out_specs=_row_spec(HID),
    out_shape=jax.ShapeDtypeStruct((NPAD, HID), jnp.float32))

_t4_call = pl.pallas_call(
    _t4_body, grid=(GRID,),
    in_specs=[_row_spec(HID), _row_spec(HID), _row_spec(L),
              _full_spec(8, HID), _full_spec(HID, HID), _full_spec(8, HID),
              _full_spec(8, HID), _full_spec(8, HID)],
    out_specs=pl.BlockSpec((RB, 1), lambda i: (i, 0)),
    out_shape=jax.ShapeDtypeStruct((N, 1), jnp.float32))


@functools.lru_cache(maxsize=1)
def _sc_kernels():
    mesh = plsc.VectorSubcoreMesh(core_axis_name="c", subcore_axis_name="s",
                                  num_cores=NC, num_subcores=NS)
    return _bin_kernel(mesh), _deg_kernel(mesh), _acc_kernel(mesh)


def kernel(x, edge_index, W1, b1, W2, b2, W3, b3, M1, mb1, M2, mb2):
    _bin_call, _deg_call, _acc_call = _sc_kernels()
    slab, counts = _bin_call(edge_index)
    deg = _deg_call(slab, counts)

    w1p = jnp.pad(W1, ((0, 5), (0, 0)))
    b1p = jnp.broadcast_to(b1.reshape(1, HID), (8, HID))
    b2p = jnp.broadcast_to(b2.reshape(1, HID), (8, HID))
    b3p = jnp.broadcast_to(b3.reshape(1, HID), (8, HID))
    mb1p = jnp.broadcast_to(mb1.reshape(1, HID), (8, HID))
    m2p = jnp.broadcast_to(M2.reshape(1, HID), (8, HID))
    mb2p = jnp.broadcast_to(mb2.reshape(1, 1), (8, HID))

    g1 = _t1_call(x, deg, w1p)
    s1 = _acc_call(slab, counts, g1)
    g2 = _tmid_call(s1, g1, deg, W2, b1p)
    s2 = _acc_call(slab, counts, g2)
    g3 = _tmid_call(s2, g2, deg, W3, b2p)
    s3 = _acc_call(slab, counts, g3)
    return _t4_call(s3, g3, deg, b3p, M1, mb1p, m2p, mb2p)
